# baseline (speedup 1.0000x reference)
import jax
import jax.numpy as jnp
from jax import lax
from jax.experimental import pallas as pl
from jax.experimental.pallas import tpu as pltpu
from jax.experimental.pallas import tpu_sc as plsc

NC = 2
NS = 16
NW = NC * NS
L = 16

N = 1_000_000
VREGS = N // L
VPW = VREGS // NW
TAIL_VREGS = VREGS - VPW * NW
PW = VPW * L
TW = TAIL_VREGS * L

U = 21
NBLK = VPW // U
BLK1 = 46
C1 = BLK1 * U * L
C2 = PW - C1
CAP = 448
DRAIN_AT = 64

INF = float("inf")


def _splat(x, dtype=jnp.float32):
    return jnp.full((L,), x, dtype=dtype)


def _merge_sorted(rv, ri, sv_desc, si_desc):
    m = sv_desc < rv
    nv = jnp.where(m, sv_desc, rv)
    ni = jnp.where(m, si_desc, ri)
    out = plsc.sort_key_val(nv, ni)
    return out[0], out[1]


def _topk_body(px_ref, py_ref, pz_ref, p1_ref,
               outv_ref, outi_ref, outx_ref, outy_ref, outz_ref,
               xb, yb, zb, p1v, candv, candi,
               stgv, stgi, stgx, stgy, stgz, dsem1, dsem2):
    wid = lax.axis_index("c") * NS + lax.axis_index("s")
    base = wid * PW

    c1 = [pltpu.async_copy(r.at[pl.ds(base, C1)], b.at[pl.ds(0, C1)], dsem1)
          for r, b in ((px_ref, xb), (py_ref, yb), (pz_ref, zb))]
    c2 = [pltpu.async_copy(r.at[pl.ds(base + C1, C2)],
                           b.at[pl.ds(C1, C2)], dsem2)
          for r, b in ((px_ref, xb), (py_ref, yb), (pz_ref, zb))]
    pltpu.sync_copy(p1_ref, p1v)

    @pl.when(wid == 0)
    def _():
        pltpu.sync_copy(px_ref.at[pl.ds(NW * PW, TW)], xb.at[pl.ds(PW, TW)])
        pltpu.sync_copy(py_ref.at[pl.ds(NW * PW, TW)], yb.at[pl.ds(PW, TW)])
        pltpu.sync_copy(pz_ref.at[pl.ds(NW * PW, TW)], zb.at[pl.ds(PW, TW)])

    for c in c1:
        c.wait()

    def _fill(j, c):
        candv[pl.ds(j * L, L)] = _splat(INF)
        return c
    lax.fori_loop(0, CAP // L, _fill, 0)

    iota = lax.iota(jnp.int32, L)
    q = p1v[...]
    qx = jnp.full((L,), q[0], dtype=jnp.float32)
    qy = jnp.full((L,), q[1], dtype=jnp.float32)
    qz = jnp.full((L,), q[2], dtype=jnp.float32)

    def drain(rv, ri, offv):
        nvregs = (offv[0] + L - 1) // L

        def body(j, c):
            rv, ri = c
            cv = candv[pl.ds(j * L, L)]
            ci = candi[pl.ds(j * L, L)]
            sv, si = plsc.sort_key_val(cv, ci, descending=True)
            rv, ri = _merge_sorted(rv, ri, sv, si)
            candv[pl.ds(j * L, L)] = _splat(INF)
            return rv, ri

        rv, ri = lax.fori_loop(0, nvregs, body, (rv, ri))
        t = jnp.full((L,), jnp.max(rv), dtype=jnp.float32)
        return rv, ri, t, jnp.zeros((L,), jnp.int32)

    def step(t, offv, g, w):
        x = xb[pl.ds(w, L)]
        y = yb[pl.ds(w, L)]
        z = zb[pl.ds(w, L)]
        dx = x - qx
        dy = y - qy
        dz = z - qz
        d = dx * dx + dy * dy + dz * dz
        m = d < t
        pos = jnp.maximum(offv + plsc.cumsum(m.astype(jnp.int32)) - 1, 0)
        plsc.store_scatter(candv, [pos], d, mask=m)
        plsc.store_scatter(candi, [pos], g, mask=m)
        return offv + plsc.all_reduce_population_count(m)

    def block(b, c):
        rv, ri, t, offv, g0 = c
        w0 = b * (U * L)
        for j in range(U):
            offv = step(t, offv, g0 + j * L, w0 + j * L)
        rv, ri, t, offv = lax.cond(
            offv[0] > DRAIN_AT,
            lambda c2: drain(c2[0], c2[1], c2[3]),
            lambda c2: c2, (rv, ri, t, offv))
        return rv, ri, t, offv, g0 + U * L

    init = (_splat(INF), jnp.zeros((L,), jnp.int32), _splat(INF),
            jnp.zeros((L,), jnp.int32), base + iota)
    carry = lax.fori_loop(0, BLK1, block, init)
    for c in c2:
        c.wait()
    carry = lax.fori_loop(BLK1, NBLK, block, carry)

    def tail(c):
        rv, ri, t, off, g0 = c
        g2 = _splat(NW * PW, jnp.int32) + iota
        for j in range(TAIL_VREGS):
            off = step(t, off, g2 + j * L, PW + j * L)
        return rv, ri, t, off, g0

    carry = lax.cond(wid == 0, tail, lambda c: c, carry)
    rv, ri, t, off, g0 = carry
    rv, ri, t, off = drain(rv, ri, off)

    rel = jnp.where(ri >= NW * PW, ri - (NW * PW) + PW, ri - base)
    rel = jnp.clip(rel, 0, PW + TW - 1)
    px = plsc.load_gather(xb, [rel])
    py = plsc.load_gather(yb, [rel])
    pz = plsc.load_gather(zb, [rel])

    stgv[...] = rv
    stgi[...] = ri
    stgx[...] = px
    stgy[...] = py
    stgz[...] = pz
    pltpu.sync_copy(stgv, outv_ref.at[pl.ds(wid * L, L)])
    pltpu.sync_copy(stgi, outi_ref.at[pl.ds(wid * L, L)])
    pltpu.sync_copy(stgx, outx_ref.at[pl.ds(wid * L, L)])
    pltpu.sync_copy(stgy, outy_ref.at[pl.ds(wid * L, L)])
    pltpu.sync_copy(stgz, outz_ref.at[pl.ds(wid * L, L)])


def _merge_body(candv_ref, candi_ref, candx_ref, candy_ref, candz_ref,
                outp_ref, outi_ref, vbuf, ibuf, xbuf, ybuf, zbuf,
                rowsb, idxb):
    wid = lax.axis_index("c") * NS + lax.axis_index("s")

    @pl.when(wid == 0)
    def _():
        pltpu.sync_copy(candv_ref, vbuf)
        pltpu.sync_copy(candi_ref, ibuf)
        pltpu.sync_copy(candx_ref, xbuf)
        pltpu.sync_copy(candy_ref, ybuf)
        pltpu.sync_copy(candz_ref, zbuf)

        iota = lax.iota(jnp.int32, L)

        rv, rp = _splat(INF), jnp.zeros((L,), jnp.int32)
        for j in range(NW):
            cv = jnp.flip(vbuf[pl.ds(j * L, L)])
            cp = jnp.flip(j * L + iota)
            rv, rp = _merge_sorted(rv, rp, cv, cp)

        ri = plsc.load_gather(ibuf, [rp])
        px = plsc.load_gather(xbuf, [rp])
        py = plsc.load_gather(ybuf, [rp])
        pz = plsc.load_gather(zbuf, [rp])

        idxb[...] = ri
        pltpu.sync_copy(idxb, outi_ref)
        plsc.store_scatter(rowsb, [iota * 3], px)
        plsc.store_scatter(rowsb, [iota * 3 + 1], py)
        plsc.store_scatter(rowsb, [iota * 3 + 2], pz)
        pltpu.sync_copy(rowsb, outp_ref)


_mesh = plsc.VectorSubcoreMesh(core_axis_name="c", subcore_axis_name="s",
                               num_cores=NC, num_subcores=NS)

_params = pltpu.CompilerParams(needs_layout_passes=False)

_topk_call = pl.kernel(
    _topk_body,
    out_type=(jax.ShapeDtypeStruct((NW * L,), jnp.float32),
              jax.ShapeDtypeStruct((NW * L,), jnp.int32),
              jax.ShapeDtypeStruct((NW * L,), jnp.float32),
              jax.ShapeDtypeStruct((NW * L,), jnp.float32),
              jax.ShapeDtypeStruct((NW * L,), jnp.float32)),
    mesh=_mesh,
    compiler_params=_params,
    scratch_types=[
        pltpu.VMEM((PW + TW,), jnp.float32),
        pltpu.VMEM((PW + TW,), jnp.float32),
        pltpu.VMEM((PW + TW,), jnp.float32),
        pltpu.VMEM((L,), jnp.float32),
        pltpu.VMEM((CAP,), jnp.float32),
        pltpu.VMEM((CAP,), jnp.int32),
        pltpu.VMEM((L,), jnp.float32),
        pltpu.VMEM((L,), jnp.int32),
        pltpu.VMEM((L,), jnp.float32),
        pltpu.VMEM((L,), jnp.float32),
        pltpu.VMEM((L,), jnp.float32),
        pltpu.SemaphoreType.DMA,
        pltpu.SemaphoreType.DMA,
    ],
)

_merge_call = pl.kernel(
    _merge_body,
    out_type=(jax.ShapeDtypeStruct((3 * L,), jnp.float32),
              jax.ShapeDtypeStruct((L,), jnp.int32)),
    mesh=_mesh,
    compiler_params=_params,
    scratch_types=[
        pltpu.VMEM((NW * L,), jnp.float32),
        pltpu.VMEM((NW * L,), jnp.int32),
        pltpu.VMEM((NW * L,), jnp.float32),
        pltpu.VMEM((NW * L,), jnp.float32),
        pltpu.VMEM((NW * L,), jnp.float32),
        pltpu.VMEM((3 * L,), jnp.float32),
        pltpu.VMEM((L,), jnp.int32),
    ],
)


def kernel(pcloud, P1, K):
    px = jnp.reshape(lax.slice(pcloud, (0, 0, 0), (1, N, 1)), (N,))
    py = jnp.reshape(lax.slice(pcloud, (0, 0, 1), (1, N, 2)), (N,))
    pz = jnp.reshape(lax.slice(pcloud, (0, 0, 2), (1, N, 3)), (N,))
    p1p = jnp.pad(jnp.asarray(P1, jnp.float32), (0, L - 3))
    cv, ci, cx, cy, cz = _topk_call(px, py, pz, p1p)
    pts, idx = _merge_call(cv, ci, cx, cy, cz)
    idx = idx + (K - 16)
    return (jnp.reshape(pts, (L, 3)), jnp.reshape(idx, (1, L)))


# speedup vs baseline: 1.1908x; 1.1908x over previous
import jax
import jax.numpy as jnp
from jax import lax
from jax.experimental import pallas as pl
from jax.experimental.pallas import tpu as pltpu
from jax.experimental.pallas import tpu_sc as plsc

NC = 2
NS = 16
NW = NC * NS
L = 16

N = 1_000_000
VREGS = N // L
VPW = VREGS // NW
TAIL_VREGS = VREGS - VPW * NW
PW = VPW * L
TW = TAIL_VREGS * L

U = 21
NBLK = VPW // U
BLK1 = 46
C1 = BLK1 * U * L
C2 = PW - C1
CAP = 448
DRAIN_AT = 64

INF = float("inf")


def _splat(x, dtype=jnp.float32):
    return jnp.full((L,), x, dtype=dtype)


def _merge_sorted(rv, ri, sv_desc, si_desc):
    m = sv_desc < rv
    nv = jnp.where(m, sv_desc, rv)
    ni = jnp.where(m, si_desc, ri)
    out = plsc.sort_key_val(nv, ni)
    return out[0], out[1]


def _topk_body(px_ref, py_ref, pz_ref, p1_ref,
               outv_ref, outi_ref, outx_ref, outy_ref, outz_ref,
               xb, yb, zb, p1v, candv, candi,
               stgv, stgi, stgx, stgy, stgz, dsem1, dsem2):
    wid = lax.axis_index("c") * NS + lax.axis_index("s")
    base = wid * PW

    c1 = [pltpu.async_copy(r.at[pl.ds(base, C1)], b.at[pl.ds(0, C1)], dsem1)
          for r, b in ((px_ref, xb), (py_ref, yb), (pz_ref, zb))]
    c2 = [pltpu.async_copy(r.at[pl.ds(base + C1, C2)],
                           b.at[pl.ds(C1, C2)], dsem2)
          for r, b in ((px_ref, xb), (py_ref, yb), (pz_ref, zb))]
    pltpu.sync_copy(p1_ref, p1v)

    @pl.when(wid == 0)
    def _():
        pltpu.sync_copy(px_ref.at[pl.ds(NW * PW, TW)], xb.at[pl.ds(PW, TW)])
        pltpu.sync_copy(py_ref.at[pl.ds(NW * PW, TW)], yb.at[pl.ds(PW, TW)])
        pltpu.sync_copy(pz_ref.at[pl.ds(NW * PW, TW)], zb.at[pl.ds(PW, TW)])

    for c in c1:
        c.wait()

    def _fill(j, c):
        candv[pl.ds(j * L, L)] = _splat(INF)
        return c
    lax.fori_loop(0, CAP // L, _fill, 0)

    iota = lax.iota(jnp.int32, L)
    q = p1v[...]
    qx = jnp.full((L,), q[0], dtype=jnp.float32)
    qy = jnp.full((L,), q[1], dtype=jnp.float32)
    qz = jnp.full((L,), q[2], dtype=jnp.float32)

    def drain(rv, ri, off):
        nvregs = (off + L - 1) // L

        def body(j, c):
            rv, ri = c
            cv = candv[pl.ds(j * L, L)]
            ci = candi[pl.ds(j * L, L)]
            sv, si = plsc.sort_key_val(cv, ci, descending=True)
            rv, ri = _merge_sorted(rv, ri, sv, si)
            candv[pl.ds(j * L, L)] = _splat(INF)
            return rv, ri

        rv, ri = lax.fori_loop(0, nvregs, body, (rv, ri))
        t = jnp.full((L,), jnp.max(rv), dtype=jnp.float32)
        return rv, ri, t, jnp.int32(0)

    def dcalc(w):
        x = xb[pl.ds(w, L)]
        y = yb[pl.ds(w, L)]
        z = zb[pl.ds(w, L)]
        dx = x - qx
        dy = y - qy
        dz = z - qz
        return dx * dx + dy * dy + dz * dz

    def step(t, off, g, w):
        d = dcalc(w)
        m = d < t
        plsc.store_compressed(candv.at[pl.ds(off, L)], d, mask=m)
        plsc.store_compressed(candi.at[pl.ds(off, L)], g, mask=m)
        return off + plsc.all_reduce_population_count(m)[0]

    def block(b, c):
        rv, ri, t, off, g0 = c
        w0 = b * (U * L)
        bm = dcalc(w0)
        for j in range(1, U):
            bm = jnp.minimum(bm, dcalc(w0 + j * L))
        hitc = plsc.all_reduce_population_count(bm < t)[0]

        def slow(c2):
            rv, ri, t, off = c2
            for j in range(U):
                off = step(t, off, g0 + j * L, w0 + j * L)
            return drain(rv, ri, off)

        rv, ri, t, off = lax.cond(hitc > 0, slow, lambda c2: c2,
                                  (rv, ri, t, off))
        return rv, ri, t, off, g0 + U * L

    init = (_splat(INF), jnp.zeros((L,), jnp.int32), _splat(INF),
            jnp.int32(0), base + iota)
    carry = lax.fori_loop(0, BLK1, block, init)
    for c in c2:
        c.wait()
    carry = lax.fori_loop(BLK1, NBLK, block, carry)

    def tail(c):
        rv, ri, t, off, g0 = c
        g2 = _splat(NW * PW, jnp.int32) + iota
        for j in range(TAIL_VREGS):
            off = step(t, off, g2 + j * L, PW + j * L)
        return rv, ri, t, off, g0

    carry = lax.cond(wid == 0, tail, lambda c: c, carry)
    rv, ri, t, off, g0 = carry
    rv, ri, t, off = drain(rv, ri, off)

    rel = jnp.where(ri >= NW * PW, ri - (NW * PW) + PW, ri - base)
    rel = jnp.clip(rel, 0, PW + TW - 1)
    px = plsc.load_gather(xb, [rel])
    py = plsc.load_gather(yb, [rel])
    pz = plsc.load_gather(zb, [rel])

    stgv[...] = rv
    stgi[...] = ri
    stgx[...] = px
    stgy[...] = py
    stgz[...] = pz
    pltpu.sync_copy(stgv, outv_ref.at[pl.ds(wid * L, L)])
    pltpu.sync_copy(stgi, outi_ref.at[pl.ds(wid * L, L)])
    pltpu.sync_copy(stgx, outx_ref.at[pl.ds(wid * L, L)])
    pltpu.sync_copy(stgy, outy_ref.at[pl.ds(wid * L, L)])
    pltpu.sync_copy(stgz, outz_ref.at[pl.ds(wid * L, L)])


def _merge_body(candv_ref, candi_ref, candx_ref, candy_ref, candz_ref,
                outp_ref, outi_ref, vbuf, ibuf, xbuf, ybuf, zbuf,
                rowsb, idxb):
    wid = lax.axis_index("c") * NS + lax.axis_index("s")

    @pl.when(wid == 0)
    def _():
        pltpu.sync_copy(candv_ref, vbuf)
        pltpu.sync_copy(candi_ref, ibuf)
        pltpu.sync_copy(candx_ref, xbuf)
        pltpu.sync_copy(candy_ref, ybuf)
        pltpu.sync_copy(candz_ref, zbuf)

        iota = lax.iota(jnp.int32, L)

        rv, rp = _splat(INF), jnp.zeros((L,), jnp.int32)
        for j in range(NW):
            cv = jnp.flip(vbuf[pl.ds(j * L, L)])
            cp = jnp.flip(j * L + iota)
            rv, rp = _merge_sorted(rv, rp, cv, cp)

        ri = plsc.load_gather(ibuf, [rp])
        px = plsc.load_gather(xbuf, [rp])
        py = plsc.load_gather(ybuf, [rp])
        pz = plsc.load_gather(zbuf, [rp])

        idxb[...] = ri
        pltpu.sync_copy(idxb, outi_ref)
        plsc.store_scatter(rowsb, [iota * 3], px)
        plsc.store_scatter(rowsb, [iota * 3 + 1], py)
        plsc.store_scatter(rowsb, [iota * 3 + 2], pz)
        pltpu.sync_copy(rowsb, outp_ref)


_mesh = plsc.VectorSubcoreMesh(core_axis_name="c", subcore_axis_name="s",
                               num_cores=NC, num_subcores=NS)

_params = pltpu.CompilerParams(needs_layout_passes=False)

_topk_call = pl.kernel(
    _topk_body,
    out_type=(jax.ShapeDtypeStruct((NW * L,), jnp.float32),
              jax.ShapeDtypeStruct((NW * L,), jnp.int32),
              jax.ShapeDtypeStruct((NW * L,), jnp.float32),
              jax.ShapeDtypeStruct((NW * L,), jnp.float32),
              jax.ShapeDtypeStruct((NW * L,), jnp.float32)),
    mesh=_mesh,
    compiler_params=_params,
    scratch_types=[
        pltpu.VMEM((PW + TW,), jnp.float32),
        pltpu.VMEM((PW + TW,), jnp.float32),
        pltpu.VMEM((PW + TW,), jnp.float32),
        pltpu.VMEM((L,), jnp.float32),
        pltpu.VMEM((CAP,), jnp.float32),
        pltpu.VMEM((CAP,), jnp.int32),
        pltpu.VMEM((L,), jnp.float32),
        pltpu.VMEM((L,), jnp.int32),
        pltpu.VMEM((L,), jnp.float32),
        pltpu.VMEM((L,), jnp.float32),
        pltpu.VMEM((L,), jnp.float32),
        pltpu.SemaphoreType.DMA,
        pltpu.SemaphoreType.DMA,
    ],
)

_merge_call = pl.kernel(
    _merge_body,
    out_type=(jax.ShapeDtypeStruct((3 * L,), jnp.float32),
              jax.ShapeDtypeStruct((L,), jnp.int32)),
    mesh=_mesh,
    compiler_params=_params,
    scratch_types=[
        pltpu.VMEM((NW * L,), jnp.float32),
        pltpu.VMEM((NW * L,), jnp.int32),
        pltpu.VMEM((NW * L,), jnp.float32),
        pltpu.VMEM((NW * L,), jnp.float32),
        pltpu.VMEM((NW * L,), jnp.float32),
        pltpu.VMEM((3 * L,), jnp.float32),
        pltpu.VMEM((L,), jnp.int32),
    ],
)


def kernel(pcloud, P1, K):
    px = jnp.reshape(lax.slice(pcloud, (0, 0, 0), (1, N, 1)), (N,))
    py = jnp.reshape(lax.slice(pcloud, (0, 0, 1), (1, N, 2)), (N,))
    pz = jnp.reshape(lax.slice(pcloud, (0, 0, 2), (1, N, 3)), (N,))
    p1p = jnp.pad(jnp.asarray(P1, jnp.float32), (0, L - 3))
    cv, ci, cx, cy, cz = _topk_call(px, py, pz, p1p)
    pts, idx = _merge_call(cv, ci, cx, cy, cz)
    idx = idx + (K - 16)
    return (jnp.reshape(pts, (L, 3)), jnp.reshape(idx, (1, L)))


# speedup vs baseline: 1.2339x; 1.0362x over previous
import jax
import jax.numpy as jnp
from jax import lax
from jax.experimental import pallas as pl
from jax.experimental.pallas import tpu as pltpu
from jax.experimental.pallas import tpu_sc as plsc

NC = 2
NS = 16
NW = NC * NS
L = 16

N = 1_000_000
U = 21
VPW1 = 987
VPW2 = 966
H1 = NW * VPW1 * L
H2N = N - H1
TAIL_VREGS = 4
TW = TAIL_VREGS * L

CAP = 448

INF = float("inf")


def _splat(x, dtype=jnp.float32):
    return jnp.full((L,), x, dtype=dtype)


def _merge_sorted(rv, ri, sv_desc, si_desc):
    m = sv_desc < rv
    nv = jnp.where(m, sv_desc, rv)
    ni = jnp.where(m, si_desc, ri)
    out = plsc.sort_key_val(nv, ni)
    return out[0], out[1]


def _make_topk(vpw, goff, has_tail):
    pw = vpw * L
    nblk = vpw // U
    blk1 = nblk // 2
    c1w = blk1 * U * L
    c2w = pw - c1w
    tail_words = TW if has_tail else 0
    tail_goff = goff + NW * pw

    def body(px_ref, py_ref, pz_ref, p1_ref,
             outv_ref, outi_ref, outx_ref, outy_ref, outz_ref,
             xb, yb, zb, p1v, candv, candi,
             stgv, stgi, stgx, stgy, stgz, dsem1, dsem2):
        wid = lax.axis_index("c") * NS + lax.axis_index("s")
        base = wid * pw

        c1 = [pltpu.async_copy(r.at[pl.ds(base, c1w)], b.at[pl.ds(0, c1w)],
                               dsem1)
              for r, b in ((px_ref, xb), (py_ref, yb), (pz_ref, zb))]
        c2 = [pltpu.async_copy(r.at[pl.ds(base + c1w, c2w)],
                               b.at[pl.ds(c1w, c2w)], dsem2)
              for r, b in ((px_ref, xb), (py_ref, yb), (pz_ref, zb))]
        pltpu.sync_copy(p1_ref, p1v)

        if has_tail:
            @pl.when(wid == 0)
            def _():
                pltpu.sync_copy(px_ref.at[pl.ds(NW * pw, TW)],
                                xb.at[pl.ds(pw, TW)])
                pltpu.sync_copy(py_ref.at[pl.ds(NW * pw, TW)],
                                yb.at[pl.ds(pw, TW)])
                pltpu.sync_copy(pz_ref.at[pl.ds(NW * pw, TW)],
                                zb.at[pl.ds(pw, TW)])

        for c in c1:
            c.wait()

        def _fill(j, c):
            candv[pl.ds(j * L, L)] = _splat(INF)
            return c
        lax.fori_loop(0, CAP // L, _fill, 0)

        iota = lax.iota(jnp.int32, L)
        q = p1v[...]
        qx = jnp.full((L,), q[0], dtype=jnp.float32)
        qy = jnp.full((L,), q[1], dtype=jnp.float32)
        qz = jnp.full((L,), q[2], dtype=jnp.float32)

        def drain(rv, ri, off):
            nvregs = (off + L - 1) // L

            def dbody(j, c):
                rv, ri = c
                cv = candv[pl.ds(j * L, L)]
                ci = candi[pl.ds(j * L, L)]
                sv, si = plsc.sort_key_val(cv, ci, descending=True)
                rv, ri = _merge_sorted(rv, ri, sv, si)
                candv[pl.ds(j * L, L)] = _splat(INF)
                return rv, ri

            rv, ri = lax.fori_loop(0, nvregs, dbody, (rv, ri))
            t = jnp.full((L,), jnp.max(rv), dtype=jnp.float32)
            return rv, ri, t, jnp.int32(0)

        def dcalc(w):
            x = xb[pl.ds(w, L)]
            y = yb[pl.ds(w, L)]
            z = zb[pl.ds(w, L)]
            dx = x - qx
            dy = y - qy
            dz = z - qz
            return dx * dx + dy * dy + dz * dz

        def step(t, off, g, w):
            d = dcalc(w)
            m = d < t
            plsc.store_compressed(candv.at[pl.ds(off, L)], d, mask=m)
            plsc.store_compressed(candi.at[pl.ds(off, L)], g, mask=m)
            return off + plsc.all_reduce_population_count(m)[0]

        def block(b, c):
            rv, ri, t, off, g0 = c
            w0 = b * (U * L)
            bm = dcalc(w0)
            for j in range(1, U):
                bm = jnp.minimum(bm, dcalc(w0 + j * L))
            hitc = plsc.all_reduce_population_count(bm < t)[0]

            def slow(c2_):
                rv, ri, t, off = c2_
                for j in range(U):
                    off = step(t, off, g0 + j * L, w0 + j * L)
                return drain(rv, ri, off)

            rv, ri, t, off = lax.cond(hitc > 0, slow, lambda c2_: c2_,
                                      (rv, ri, t, off))
            return rv, ri, t, off, g0 + U * L

        init = (_splat(INF), jnp.zeros((L,), jnp.int32), _splat(INF),
                jnp.int32(0), goff + base + iota)
        carry = lax.fori_loop(0, blk1, block, init)
        for c in c2:
            c.wait()
        carry = lax.fori_loop(blk1, nblk, block, carry)

        if has_tail:
            def tail(c):
                rv, ri, t, off, g0 = c
                g2 = _splat(tail_goff, jnp.int32) + iota
                for j in range(TAIL_VREGS):
                    off = step(t, off, g2 + j * L, pw + j * L)
                return rv, ri, t, off, g0

            carry = lax.cond(wid == 0, tail, lambda c: c, carry)
        rv, ri, t, off, g0 = carry
        rv, ri, t, off = drain(rv, ri, off)

        rel = jnp.where(ri >= tail_goff, ri - tail_goff + pw,
                        ri - (goff + base))
        rel = jnp.clip(rel, 0, pw + tail_words - 1)
        px = plsc.load_gather(xb, [rel])
        py = plsc.load_gather(yb, [rel])
        pz = plsc.load_gather(zb, [rel])

        stgv[...] = rv
        stgi[...] = ri
        stgx[...] = px
        stgy[...] = py
        stgz[...] = pz
        pltpu.sync_copy(stgv, outv_ref.at[pl.ds(wid * L, L)])
        pltpu.sync_copy(stgi, outi_ref.at[pl.ds(wid * L, L)])
        pltpu.sync_copy(stgx, outx_ref.at[pl.ds(wid * L, L)])
        pltpu.sync_copy(stgy, outy_ref.at[pl.ds(wid * L, L)])
        pltpu.sync_copy(stgz, outz_ref.at[pl.ds(wid * L, L)])

    return pl.kernel(
        body,
        out_type=(jax.ShapeDtypeStruct((NW * L,), jnp.float32),
                  jax.ShapeDtypeStruct((NW * L,), jnp.int32),
                  jax.ShapeDtypeStruct((NW * L,), jnp.float32),
                  jax.ShapeDtypeStruct((NW * L,), jnp.float32),
                  jax.ShapeDtypeStruct((NW * L,), jnp.float32)),
        mesh=_mesh,
        compiler_params=_params,
        scratch_types=[
            pltpu.VMEM((pw + tail_words,), jnp.float32),
            pltpu.VMEM((pw + tail_words,), jnp.float32),
            pltpu.VMEM((pw + tail_words,), jnp.float32),
            pltpu.VMEM((L,), jnp.float32),
            pltpu.VMEM((CAP,), jnp.float32),
            pltpu.VMEM((CAP,), jnp.int32),
            pltpu.VMEM((L,), jnp.float32),
            pltpu.VMEM((L,), jnp.int32),
            pltpu.VMEM((L,), jnp.float32),
            pltpu.VMEM((L,), jnp.float32),
            pltpu.VMEM((L,), jnp.float32),
            pltpu.SemaphoreType.DMA,
            pltpu.SemaphoreType.DMA,
        ],
    )


def _merge_body(candv_ref, candi_ref, candx_ref, candy_ref, candz_ref,
                outp_ref, outi_ref, vbuf, ibuf, xbuf, ybuf, zbuf,
                rowsb, idxb):
    wid = lax.axis_index("c") * NS + lax.axis_index("s")

    @pl.when(wid == 0)
    def _():
        pltpu.sync_copy(candv_ref, vbuf)
        pltpu.sync_copy(candi_ref, ibuf)
        pltpu.sync_copy(candx_ref, xbuf)
        pltpu.sync_copy(candy_ref, ybuf)
        pltpu.sync_copy(candz_ref, zbuf)

        iota = lax.iota(jnp.int32, L)

        rv, rp = _splat(INF), jnp.zeros((L,), jnp.int32)
        for j in range(2 * NW):
            cv = jnp.flip(vbuf[pl.ds(j * L, L)])
            cp = jnp.flip(j * L + iota)
            rv, rp = _merge_sorted(rv, rp, cv, cp)

        ri = plsc.load_gather(ibuf, [rp])
        px = plsc.load_gather(xbuf, [rp])
        py = plsc.load_gather(ybuf, [rp])
        pz = plsc.load_gather(zbuf, [rp])

        idxb[...] = ri
        pltpu.sync_copy(idxb, outi_ref)
        plsc.store_scatter(rowsb, [iota * 3], px)
        plsc.store_scatter(rowsb, [iota * 3 + 1], py)
        plsc.store_scatter(rowsb, [iota * 3 + 2], pz)
        pltpu.sync_copy(rowsb, outp_ref)


_mesh = plsc.VectorSubcoreMesh(core_axis_name="c", subcore_axis_name="s",
                               num_cores=NC, num_subcores=NS)

_params = pltpu.CompilerParams(needs_layout_passes=False)

_topk1 = _make_topk(VPW1, 0, False)
_topk2 = _make_topk(VPW2, H1, True)

_merge_call = pl.kernel(
    _merge_body,
    out_type=(jax.ShapeDtypeStruct((3 * L,), jnp.float32),
              jax.ShapeDtypeStruct((L,), jnp.int32)),
    mesh=_mesh,
    compiler_params=_params,
    scratch_types=[
        pltpu.VMEM((2 * NW * L,), jnp.float32),
        pltpu.VMEM((2 * NW * L,), jnp.int32),
        pltpu.VMEM((2 * NW * L,), jnp.float32),
        pltpu.VMEM((2 * NW * L,), jnp.float32),
        pltpu.VMEM((2 * NW * L,), jnp.float32),
        pltpu.VMEM((3 * L,), jnp.float32),
        pltpu.VMEM((L,), jnp.int32),
    ],
)


def kernel(pcloud, P1, K):
    p1p = jnp.pad(jnp.asarray(P1, jnp.float32), (0, L - 3))
    px1 = jnp.reshape(lax.slice(pcloud, (0, 0, 0), (1, H1, 1)), (H1,))
    py1 = jnp.reshape(lax.slice(pcloud, (0, 0, 1), (1, H1, 2)), (H1,))
    pz1 = jnp.reshape(lax.slice(pcloud, (0, 0, 2), (1, H1, 3)), (H1,))
    px2 = jnp.reshape(lax.slice(pcloud, (0, H1, 0), (1, N, 1)), (H2N,))
    py2 = jnp.reshape(lax.slice(pcloud, (0, H1, 1), (1, N, 2)), (H2N,))
    pz2 = jnp.reshape(lax.slice(pcloud, (0, H1, 2), (1, N, 3)), (H2N,))
    o1 = _topk1(px1, py1, pz1, p1p)
    o2 = _topk2(px2, py2, pz2, p1p)
    cands = [jnp.concatenate([a, b]) for a, b in zip(o1, o2)]
    pts, idx = _merge_call(*cands)
    idx = idx + (K - 16)
    return (jnp.reshape(pts, (L, 3)), jnp.reshape(idx, (1, L)))
